# baseline (speedup 1.0000x reference)
import functools

import jax
import jax.numpy as jnp
from jax import lax
from jax.experimental import pallas as pl
from jax.experimental.pallas import tpu as pltpu
from jax.experimental.pallas import tpu_sc as plsc

_DIM = 256
_K = 8192
_BETA = 0.25

_TN = 256
_TK = 1024
_KT = _K // _TK

_BIG_I32 = 2 ** 30


def _vq_tc_body(z_ref, wt_ref, zsq_ref, wsq_ref, idx_ref, m_ref):
    z = z_ref[...]
    zsq = zsq_ref[...]

    m = jnp.full((_TN, 1), jnp.inf, dtype=jnp.float32)
    a = jnp.zeros((_TN, 1), dtype=jnp.int32)
    for k in range(_KT):
        wt = wt_ref[:, pl.ds(k * _TK, _TK)]
        wsq = wsq_ref[0:1, pl.ds(k * _TK, _TK)]
        p = lax.dot_general(z, wt, (((1,), (0,)), ((), ())),
                            preferred_element_type=jnp.float32)
        d2 = (zsq + wsq) - 2.0 * p
        dist = jnp.sqrt(jnp.maximum(d2, 0.0))
        mt = jnp.min(dist, axis=1, keepdims=True)
        col = lax.broadcasted_iota(jnp.int32, (_TN, _TK), 1) + (k * _TK)
        at = jnp.min(jnp.where(dist == mt, col, _BIG_I32), axis=1, keepdims=True)
        upd = mt < m
        a = jnp.where(upd, at, a)
        m = jnp.where(upd, mt, m)

    idx_ref[...] = a
    m_ref[...] = m


def _nearest_codes(z_flat, wt, zsq, wsq_b):
    n_tokens = z_flat.shape[0]
    grid = (n_tokens // _TN,)
    return pl.pallas_call(
        _vq_tc_body,
        grid=grid,
        in_specs=[
            pl.BlockSpec((_TN, _DIM), lambda n: (n, 0)),
            pl.BlockSpec((_DIM, _K), lambda n: (0, 0)),
            pl.BlockSpec((_TN, 1), lambda n: (n, 0)),
            pl.BlockSpec((8, _K), lambda n: (0, 0)),
        ],
        out_specs=[
            pl.BlockSpec((_TN, 1), lambda n: (n, 0)),
            pl.BlockSpec((_TN, 1), lambda n: (n, 0)),
        ],
        out_shape=[
            jax.ShapeDtypeStruct((n_tokens, 1), jnp.int32),
            jax.ShapeDtypeStruct((n_tokens, 1), jnp.float32),
        ],
    )(z_flat, wt, zsq, wsq_b)


_GATHER_CHUNK = 128


def _gather_rows(table, idx):
    n = idx.shape[0]
    info = plsc.get_sparse_core_info()
    nw = info.num_cores * info.num_subcores
    per_w = n // nw
    chunks = per_w // _GATHER_CHUNK
    mesh = plsc.VectorSubcoreMesh(core_axis_name="c", subcore_axis_name="s")

    @functools.partial(
        pl.kernel,
        mesh=mesh,
        out_type=jax.ShapeDtypeStruct((n, _DIM), jnp.float32),
        scratch_types=[
            pltpu.VMEM((_GATHER_CHUNK,), jnp.int32),
            pltpu.VMEM((_GATHER_CHUNK, _DIM), jnp.float32),
            pltpu.SemaphoreType.DMA,
        ],
    )
    def k(table_hbm, idx_hbm, out_hbm, idx_v, rows_v, sem):
        wid = lax.axis_index("s") * info.num_cores + lax.axis_index("c")
        base = wid * per_w
        for c in range(chunks):
            off = base + c * _GATHER_CHUNK
            pltpu.sync_copy(idx_hbm.at[pl.ds(off, _GATHER_CHUNK)], idx_v)
            pltpu.async_copy(table_hbm.at[idx_v], rows_v, sem).wait()
            pltpu.sync_copy(rows_v, out_hbm.at[pl.ds(off, _GATHER_CHUNK)])

    return k(table, idx)


def kernel(z, W):
    b, dim, t = z.shape
    n_tokens = b * t
    z_flat = jnp.transpose(z, (0, 2, 1)).reshape(-1, dim)
    zsq = jnp.sum(z_flat * z_flat, axis=1, keepdims=True)
    wsq = jnp.sum(W * W, axis=1)
    wsq_b = jnp.broadcast_to(wsq[None, :], (8, _K))
    wt = W.T

    idx2d, mmin = _nearest_codes(z_flat, wt, zsq, wsq_b)
    idx_flat = idx2d[:, 0]

    z_q_flat = _gather_rows(W, idx_flat)

    z_q = jnp.transpose(z_q_flat.reshape(b, t, dim), (0, 2, 1))
    z_q_st = z + lax.stop_gradient(z_q - z)
    loss = (1.0 + _BETA) * jnp.sum(mmin * mmin) / jnp.float32(n_tokens * dim)
    indices = idx_flat.reshape(b, t)
    return (z_q_st, indices, loss)


# speedup vs baseline: 1.1735x; 1.1735x over previous
import functools

import jax
import jax.numpy as jnp
from jax import lax
from jax.experimental import pallas as pl
from jax.experimental.pallas import tpu as pltpu
from jax.experimental.pallas import tpu_sc as plsc

_DIM = 256
_K = 8192
_BETA = 0.25

_TN = 256
_TK = 1024
_KT = _K // _TK

_BIG_I32 = 2 ** 30


def _vq_tc_body(z_ref, wt2_ref, zsq_ref, wsq_ref, idx_ref, m_ref):
    z = z_ref[...]
    zsq = zsq_ref[...]

    macc = jnp.full((_TN, _TK), jnp.inf, dtype=jnp.float32)
    tacc = jnp.zeros((_TN, _TK), dtype=jnp.float32)
    for k in range(_KT):
        wt2 = wt2_ref[:, pl.ds(k * _TK, _TK)]
        wsq = wsq_ref[0:1, pl.ds(k * _TK, _TK)]
        p2 = lax.dot_general(z, wt2, (((1,), (0,)), ((), ())),
                             preferred_element_type=jnp.float32)
        d2 = (zsq + wsq) - p2
        dist = jnp.sqrt(jnp.maximum(d2, 0.0))
        upd = dist < macc
        tacc = jnp.where(upd, float(k), tacc)
        macc = jnp.where(upd, dist, macc)

    m = jnp.min(macc, axis=1, keepdims=True)
    col = tacc * float(_TK) + lax.broadcasted_iota(
        jnp.int32, (_TN, _TK), 1).astype(jnp.float32)
    a = jnp.min(jnp.where(macc == m, col, float(_BIG_I32)), axis=1, keepdims=True)
    idx_ref[...] = a.astype(jnp.int32)
    m_ref[...] = m


def _nearest_codes(z_flat, wt, zsq, wsq_b):
    n_tokens = z_flat.shape[0]
    grid = (n_tokens // _TN,)
    return pl.pallas_call(
        _vq_tc_body,
        grid=grid,
        in_specs=[
            pl.BlockSpec((_TN, _DIM), lambda n: (n, 0)),
            pl.BlockSpec((_DIM, _K), lambda n: (0, 0)),
            pl.BlockSpec((_TN, 1), lambda n: (n, 0)),
            pl.BlockSpec((8, _K), lambda n: (0, 0)),
        ],
        out_specs=[
            pl.BlockSpec((_TN, 1), lambda n: (n, 0)),
            pl.BlockSpec((_TN, 1), lambda n: (n, 0)),
        ],
        out_shape=[
            jax.ShapeDtypeStruct((n_tokens, 1), jnp.int32),
            jax.ShapeDtypeStruct((n_tokens, 1), jnp.float32),
        ],
    )(z_flat, wt, zsq, wsq_b)


_GATHER_CHUNK = 128


def _gather_rows(table, idx):
    n = idx.shape[0]
    info = plsc.get_sparse_core_info()
    nw = info.num_cores * info.num_subcores
    per_w = n // nw
    chunks = per_w // _GATHER_CHUNK
    mesh = plsc.VectorSubcoreMesh(core_axis_name="c", subcore_axis_name="s")

    @functools.partial(
        pl.kernel,
        mesh=mesh,
        out_type=jax.ShapeDtypeStruct((n, _DIM), jnp.float32),
        scratch_types=[
            pltpu.VMEM((_GATHER_CHUNK,), jnp.int32),
            pltpu.VMEM((_GATHER_CHUNK, _DIM), jnp.float32),
            pltpu.SemaphoreType.DMA,
        ],
    )
    def k(table_hbm, idx_hbm, out_hbm, idx_v, rows_v, sem):
        wid = lax.axis_index("s") * info.num_cores + lax.axis_index("c")
        base = wid * per_w
        for c in range(chunks):
            off = base + c * _GATHER_CHUNK
            pltpu.sync_copy(idx_hbm.at[pl.ds(off, _GATHER_CHUNK)], idx_v)
            pltpu.async_copy(table_hbm.at[idx_v], rows_v, sem).wait()
            pltpu.sync_copy(rows_v, out_hbm.at[pl.ds(off, _GATHER_CHUNK)])

    return k(table, idx)


def kernel(z, W):
    b, dim, t = z.shape
    n_tokens = b * t
    z_flat = jnp.transpose(z, (0, 2, 1)).reshape(-1, dim)
    zsq = jnp.sum(z_flat * z_flat, axis=1, keepdims=True)
    wsq = jnp.sum(W * W, axis=1)
    wsq_b = jnp.broadcast_to(wsq[None, :], (8, _K))
    wt2 = (W + W).T

    idx2d, mmin = _nearest_codes(z_flat, wt2, zsq, wsq_b)
    idx_flat = idx2d[:, 0]

    z_q_flat = _gather_rows(W, idx_flat)

    z_q = jnp.transpose(z_q_flat.reshape(b, t, dim), (0, 2, 1))
    z_q_st = z + lax.stop_gradient(z_q - z)
    loss = (1.0 + _BETA) * jnp.sum(mmin * mmin) / jnp.float32(n_tokens * dim)
    indices = idx_flat.reshape(b, t)
    return (z_q_st, indices, loss)
